# baseline (speedup 1.0000x reference)
import functools

import jax
import jax.numpy as jnp
from jax import lax
from jax.experimental import pallas as pl
from jax.experimental.pallas import tpu as pltpu
from jax.experimental.pallas import tpu_sc as plsc

N_SPECIES = 5
N_CODONS = 64
B = 4096
L = 2048
LW = L // 4

_info = plsc.get_sparse_core_info()
NC, NS, LANES = _info.num_cores, _info.num_subcores, _info.num_lanes
NW = NC * NS
RPW = B // NW
NCHUNK = 4
CROWS = RPW // NCHUNK
SUBG = CROWS // LANES


def _cai_sc(pk_hbm, sid_hbm, tbl_hbm, out_hbm,
            b0, b1, b2, b3, tbl_v, sid_v, out_v, s0, s1, s2, s3):
    wid = lax.axis_index("s") * NC + lax.axis_index("c")
    base_row = wid * RPW

    pltpu.sync_copy(tbl_hbm, tbl_v)
    pltpu.sync_copy(sid_hbm.at[pl.ds(base_row, RPW)], sid_v)

    bufs = (b0, b1, b2, b3)
    sems = (s0, s1, s2, s3)
    copies = []
    for c in range(NCHUNK):
        off = (base_row + c * CROWS) * LW
        cp = pltpu.make_async_copy(
            pk_hbm.at[pl.ds(off, CROWS * LW)], bufs[c], sems[c])
        cp.start()
        copies.append(cp)

    row16 = lax.iota(jnp.int32, LANES)

    for c in range(NCHUNK):
        copies[c].wait()

    for c in range(NCHUNK):
        for sub in range(SUBG):
            lg = c * SUBG + sub
            sidv = sid_v[pl.ds(lg * LANES, LANES)]
            sb = sidv * 128
            base = sub * (LANES * LW)

            acc = jnp.zeros((LANES,), jnp.float32)
            cnt = jnp.zeros((LANES,), jnp.int32)
            for q in range(4):
                def body(i, carry):
                    acc, cacc = carry
                    w = bufs[c][pl.ds(base + i * LANES, LANES)]
                    cacc = cacc + ((w >> 6) & 0x01010101)
                    for j in range(4):
                        t = (w >> (8 * j)) if j else w
                        g = plsc.load_gather(tbl_v, [(t & 0x7F) + sb])
                        acc = acc + g
                    return acc, cacc

                acc, cacc = lax.fori_loop(
                    q * (LW // 4), (q + 1) * (LW // 4), body,
                    (acc, jnp.zeros((LANES,), jnp.int32)))
                cnt = (cnt + (cacc & 0xFF) + ((cacc >> 8) & 0xFF)
                       + ((cacc >> 16) & 0xFF) + (cacc >> 24))

            cnt_f = jnp.maximum(cnt.astype(jnp.float32), 1.0)
            out_v[pl.ds(lg * LANES, LANES)] = jnp.exp(acc / cnt_f)

    pltpu.sync_copy(out_v, out_hbm.at[pl.ds(base_row, RPW)])


@jax.jit
def kernel(codon_ids, species_ids, mask, weight_matrix):
    logw = jnp.log(jnp.maximum(weight_matrix, 1e-8))
    tbl = jnp.concatenate(
        [jnp.zeros((N_SPECIES, N_CODONS), jnp.float32), logw], axis=1)

    x = codon_ids | (mask.astype(jnp.int32) << 6)
    pw = (x[:, 0:LW] | (x[:, LW:2 * LW] << 8)
          | (x[:, 2 * LW:3 * LW] << 16)
          | (x[:, 3 * LW:] << 24))
    packed_words = pw.reshape(B // LANES, LANES, LW).transpose(
        0, 2, 1).reshape(-1)

    mesh = plsc.VectorSubcoreMesh(core_axis_name="c", subcore_axis_name="s")
    run = pl.kernel(
        _cai_sc,
        mesh=mesh,
        compiler_params=pltpu.CompilerParams(needs_layout_passes=False),
        out_type=jax.ShapeDtypeStruct((B,), jnp.float32),
        scratch_types=[
            pltpu.VMEM((CROWS * LW,), jnp.int32),
            pltpu.VMEM((CROWS * LW,), jnp.int32),
            pltpu.VMEM((CROWS * LW,), jnp.int32),
            pltpu.VMEM((CROWS * LW,), jnp.int32),
            pltpu.VMEM((N_SPECIES * 128,), jnp.float32),
            pltpu.VMEM((RPW,), jnp.int32),
            pltpu.VMEM((RPW,), jnp.float32),
            pltpu.SemaphoreType.DMA,
            pltpu.SemaphoreType.DMA,
            pltpu.SemaphoreType.DMA,
            pltpu.SemaphoreType.DMA,
        ],
    )
    return run(packed_words, species_ids, tbl.reshape(-1))


# speedup vs baseline: 1.3935x; 1.3935x over previous
import functools

import jax
import jax.numpy as jnp
from jax import lax
from jax.experimental import pallas as pl
from jax.experimental.pallas import tpu as pltpu
from jax.experimental.pallas import tpu_sc as plsc

N_SPECIES = 5
N_CODONS = 64
B = 4096
L = 2048
LW = L // 4

_info = plsc.get_sparse_core_info()
NC, NS, LANES = _info.num_cores, _info.num_subcores, _info.num_lanes
NW = NC * NS
RPW = B // NW
NCHUNK = 4
CROWS = RPW // NCHUNK
SUBG = CROWS // LANES


def _cai_sc(pk_hbm, sid_hbm, tbl_hbm, out_hbm,
            b0, b1, b2, b3, tbl_v, sid_v, out_v, sum_v, cnt_v,
            s0, s1, s2, s3):
    wid = lax.axis_index("s") * NC + lax.axis_index("c")
    base_row = wid * RPW

    pltpu.sync_copy(tbl_hbm, tbl_v)
    pltpu.sync_copy(sid_hbm.at[pl.ds(base_row, RPW)], sid_v)

    bufs = (b0, b1, b2, b3)
    sems = (s0, s1, s2, s3)
    copies = []
    for c in range(NCHUNK):
        off = (base_row + c * CROWS) * LW
        cp = pltpu.make_async_copy(
            pk_hbm.at[pl.ds(off, CROWS * LW)], bufs[c], sems[c])
        cp.start()
        copies.append(cp)

    row16 = lax.iota(jnp.int32, LANES)

    for c in range(NCHUNK):
        copies[c].wait()

    m15 = row16 == (LANES - 1)

    for c in range(NCHUNK):
        def row_body(r, _):
            row = c * CROWS + r
            rowv = jnp.full((LANES,), row, jnp.int32)
            sbv = plsc.load_gather(sid_v, [rowv]) * 128
            off = r * LW

            def body(i, carry):
                acc, cacc = carry
                w = bufs[c][pl.ds(off + i * LANES, LANES)]
                cacc = cacc + ((w >> 6) & 0x01010101)
                for j in range(4):
                    t = (w >> (8 * j)) if j else w
                    acc = acc + plsc.load_gather(tbl_v, [(t & 0x7F) + sbv])
                return acc, cacc

            acc, cacc = lax.fori_loop(
                0, LW // LANES, body,
                (jnp.zeros((LANES,), jnp.float32),
                 jnp.zeros((LANES,), jnp.int32)))

            plsc.store_scatter(sum_v, [rowv], plsc.cumsum(acc), mask=m15)
            cbytes = ((cacc & 0xFF) + ((cacc >> 8) & 0xFF)
                      + ((cacc >> 16) & 0xFF) + (cacc >> 24))
            plsc.store_scatter(cnt_v, [rowv], plsc.cumsum(cbytes), mask=m15)
            return 0

        lax.fori_loop(0, CROWS, row_body, 0)

    for g in range(RPW // LANES):
        s = sum_v[pl.ds(g * LANES, LANES)]
        n = cnt_v[pl.ds(g * LANES, LANES)]
        out_v[pl.ds(g * LANES, LANES)] = jnp.exp(
            s / jnp.maximum(n.astype(jnp.float32), 1.0))

    pltpu.sync_copy(out_v, out_hbm.at[pl.ds(base_row, RPW)])


@jax.jit
def kernel(codon_ids, species_ids, mask, weight_matrix):
    logw = jnp.log(jnp.maximum(weight_matrix, 1e-8))
    tbl = jnp.concatenate(
        [jnp.zeros((N_SPECIES, N_CODONS), jnp.float32), logw], axis=1)

    x = codon_ids | (mask.astype(jnp.int32) << 6)
    packed_words = (x[:, 0:LW] | (x[:, LW:2 * LW] << 8)
                    | (x[:, 2 * LW:3 * LW] << 16)
                    | (x[:, 3 * LW:] << 24)).reshape(-1)

    mesh = plsc.VectorSubcoreMesh(core_axis_name="c", subcore_axis_name="s")
    run = pl.kernel(
        _cai_sc,
        mesh=mesh,
        compiler_params=pltpu.CompilerParams(needs_layout_passes=False),
        out_type=jax.ShapeDtypeStruct((B,), jnp.float32),
        scratch_types=[
            pltpu.VMEM((CROWS * LW,), jnp.int32),
            pltpu.VMEM((CROWS * LW,), jnp.int32),
            pltpu.VMEM((CROWS * LW,), jnp.int32),
            pltpu.VMEM((CROWS * LW,), jnp.int32),
            pltpu.VMEM((N_SPECIES * 128,), jnp.float32),
            pltpu.VMEM((RPW,), jnp.int32),
            pltpu.VMEM((RPW,), jnp.float32),
            pltpu.VMEM((RPW,), jnp.float32),
            pltpu.VMEM((RPW,), jnp.int32),
            pltpu.SemaphoreType.DMA,
            pltpu.SemaphoreType.DMA,
            pltpu.SemaphoreType.DMA,
            pltpu.SemaphoreType.DMA,
        ],
    )
    return run(packed_words, species_ids, tbl.reshape(-1))


# speedup vs baseline: 1.5546x; 1.1156x over previous
import functools

import jax
import jax.numpy as jnp
from jax import lax
from jax.experimental import pallas as pl
from jax.experimental.pallas import tpu as pltpu
from jax.experimental.pallas import tpu_sc as plsc

N_SPECIES = 5
N_CODONS = 64
B = 4096
L = 2048
LW = L // 4

_info = plsc.get_sparse_core_info()
NC, NS, LANES = _info.num_cores, _info.num_subcores, _info.num_lanes
NW = NC * NS
RPW = B // NW
NCHUNK = 4
CROWS = RPW // NCHUNK
SUBG = CROWS // LANES


def _cai_sc(pk_hbm, sid_hbm, tbl_hbm, out_hbm,
            b0, b1, b2, b3, tbl_v, sid_v, out_v, sum_v, cnt_v,
            s0, s1, s2, s3):
    wid = lax.axis_index("s") * NC + lax.axis_index("c")
    base_row = wid * RPW

    pltpu.sync_copy(tbl_hbm, tbl_v)
    pltpu.sync_copy(sid_hbm.at[pl.ds(base_row, RPW)], sid_v)

    bufs = (b0, b1, b2, b3)
    sems = (s0, s1, s2, s3)
    copies = []
    for c in range(NCHUNK):
        cp = pltpu.make_async_copy(
            pk_hbm.at[pl.ds(base_row + c * CROWS, CROWS)], bufs[c], sems[c])
        cp.start()
        copies.append(cp)

    row16 = lax.iota(jnp.int32, LANES)

    for c in range(NCHUNK):
        copies[c].wait()

    m15 = row16 == (LANES - 1)

    for c in range(NCHUNK):
        def row_body(r, _):
            row = c * CROWS + r
            rowv = jnp.full((LANES,), row, jnp.int32)
            sbv = plsc.load_gather(sid_v, [rowv]) * 128

            def body(i, carry):
                acc, cacc = carry
                w = bufs[c][r, pl.ds(i * LANES, LANES)]
                cacc = cacc + ((w >> 6) & 0x01010101)
                for j in range(4):
                    t = (w >> (8 * j)) if j else w
                    acc = acc + plsc.load_gather(tbl_v, [(t & 0x7F) + sbv])
                return acc, cacc

            acc, cacc = lax.fori_loop(
                0, LW // LANES, body,
                (jnp.zeros((LANES,), jnp.float32),
                 jnp.zeros((LANES,), jnp.int32)))

            plsc.store_scatter(sum_v, [rowv], plsc.cumsum(acc), mask=m15)
            cbytes = ((cacc & 0xFF) + ((cacc >> 8) & 0xFF)
                      + ((cacc >> 16) & 0xFF) + (cacc >> 24))
            plsc.store_scatter(cnt_v, [rowv], plsc.cumsum(cbytes), mask=m15)
            return 0

        lax.fori_loop(0, CROWS, row_body, 0)

    for g in range(RPW // LANES):
        s = sum_v[pl.ds(g * LANES, LANES)]
        n = cnt_v[pl.ds(g * LANES, LANES)]
        out_v[pl.ds(g * LANES, LANES)] = jnp.exp(
            s / jnp.maximum(n.astype(jnp.float32), 1.0))

    pltpu.sync_copy(out_v, out_hbm.at[pl.ds(base_row, RPW)])


@jax.jit
def kernel(codon_ids, species_ids, mask, weight_matrix):
    logw = jnp.log(jnp.maximum(weight_matrix, 1e-8))
    tbl = jnp.concatenate(
        [jnp.zeros((N_SPECIES, N_CODONS), jnp.float32), logw], axis=1)

    x = codon_ids | (mask.astype(jnp.int32) << 6)
    packed_words = (x[:, 0:LW] | (x[:, LW:2 * LW] << 8)
                    | (x[:, 2 * LW:3 * LW] << 16)
                    | (x[:, 3 * LW:] << 24))

    mesh = plsc.VectorSubcoreMesh(core_axis_name="c", subcore_axis_name="s")
    run = pl.kernel(
        _cai_sc,
        mesh=mesh,
        compiler_params=pltpu.CompilerParams(needs_layout_passes=False),
        out_type=jax.ShapeDtypeStruct((B,), jnp.float32),
        scratch_types=[
            pltpu.VMEM((CROWS, LW), jnp.int32),
            pltpu.VMEM((CROWS, LW), jnp.int32),
            pltpu.VMEM((CROWS, LW), jnp.int32),
            pltpu.VMEM((CROWS, LW), jnp.int32),
            pltpu.VMEM((N_SPECIES * 128,), jnp.float32),
            pltpu.VMEM((RPW,), jnp.int32),
            pltpu.VMEM((RPW,), jnp.float32),
            pltpu.VMEM((RPW,), jnp.float32),
            pltpu.VMEM((RPW,), jnp.int32),
            pltpu.SemaphoreType.DMA,
            pltpu.SemaphoreType.DMA,
            pltpu.SemaphoreType.DMA,
            pltpu.SemaphoreType.DMA,
        ],
    )
    return run(packed_words, species_ids, tbl.reshape(-1))


# speedup vs baseline: 2.0984x; 1.3498x over previous
import functools

import jax
import jax.numpy as jnp
from jax import lax
from jax.experimental import pallas as pl
from jax.experimental.pallas import tpu as pltpu
from jax.experimental.pallas import tpu_sc as plsc

N_SPECIES = 5
N_CODONS = 64
B = 4096
L = 2048
LW = L // 4

_info = plsc.get_sparse_core_info()
NC, NS, LANES = _info.num_cores, _info.num_subcores, _info.num_lanes
NW = NC * NS
RPW = B // NW
NCHUNK = 4
CROWS = RPW // NCHUNK
SUBG = CROWS // LANES


def _cai_sc(pk_hbm, sid_hbm, tbl_hbm, out_hbm,
            b0, b1, b2, b3, tbl_v, sid_v, out_v, sum_v, cnt_v,
            s0, s1, s2, s3):
    wid = lax.axis_index("s") * NC + lax.axis_index("c")
    base_row = wid * RPW

    pltpu.sync_copy(tbl_hbm, tbl_v)
    pltpu.sync_copy(sid_hbm.at[pl.ds(base_row, RPW)], sid_v)

    bufs = (b0, b1, b2, b3)
    sems = (s0, s1, s2, s3)
    copies = []
    for c in range(NCHUNK):
        cp = pltpu.make_async_copy(
            pk_hbm.at[pl.ds(base_row + c * CROWS, CROWS)], bufs[c], sems[c])
        cp.start()
        copies.append(cp)

    row16 = lax.iota(jnp.int32, LANES)

    for c in range(NCHUNK):
        copies[c].wait()

    m15 = row16 == (LANES - 1)

    for c in range(NCHUNK):
        def row_body(r, _):
            row = c * CROWS + r
            rowv = jnp.full((LANES,), row, jnp.int32)
            sbv = plsc.load_gather(sid_v, [rowv]) * 128

            def body(i, carry):
                acc, cacc = carry
                w = bufs[c][r, pl.ds(i * LANES, LANES)]
                cacc = cacc + ((w >> 6) & 0x01010101)
                for j in range(4):
                    t = (w >> (8 * j)) if j else w
                    acc = acc + plsc.load_gather(tbl_v, [(t & 0x7F) + sbv])
                return acc, cacc

            acc, cacc = lax.fori_loop(
                0, LW // LANES, body,
                (jnp.zeros((LANES,), jnp.float32),
                 jnp.zeros((LANES,), jnp.int32)))

            plsc.store_scatter(sum_v, [rowv], plsc.cumsum(acc), mask=m15)
            cbytes = ((cacc & 0xFF) + ((cacc >> 8) & 0xFF)
                      + ((cacc >> 16) & 0xFF) + (cacc >> 24))
            plsc.store_scatter(cnt_v, [rowv], plsc.cumsum(cbytes), mask=m15)
            return 0

        lax.fori_loop(0, CROWS, row_body, 0)

    for g in range(RPW // LANES):
        s = sum_v[pl.ds(g * LANES, LANES)]
        n = cnt_v[pl.ds(g * LANES, LANES)]
        out_v[pl.ds(g * LANES, LANES)] = jnp.exp(
            s / jnp.maximum(n.astype(jnp.float32), 1.0))

    pltpu.sync_copy(out_v, out_hbm.at[pl.ds(base_row, RPW)])


@jax.jit
def kernel(codon_ids, species_ids, mask, weight_matrix):
    logw = jnp.log(jnp.maximum(weight_matrix, 1e-8))
    tbl = jnp.concatenate(
        [jnp.zeros((N_SPECIES, N_CODONS), jnp.float32), logw], axis=1)

    def byte(j):
        cj = codon_ids[:, j * LW:(j + 1) * LW]
        mj = mask[:, j * LW:(j + 1) * LW].astype(jnp.int32)
        return (cj | (mj << 6)) << (8 * j)

    packed_words = byte(0) | byte(1) | byte(2) | byte(3)

    mesh = plsc.VectorSubcoreMesh(core_axis_name="c", subcore_axis_name="s")
    run = pl.kernel(
        _cai_sc,
        mesh=mesh,
        compiler_params=pltpu.CompilerParams(needs_layout_passes=False),
        out_type=jax.ShapeDtypeStruct((B,), jnp.float32),
        scratch_types=[
            pltpu.VMEM((CROWS, LW), jnp.int32),
            pltpu.VMEM((CROWS, LW), jnp.int32),
            pltpu.VMEM((CROWS, LW), jnp.int32),
            pltpu.VMEM((CROWS, LW), jnp.int32),
            pltpu.VMEM((N_SPECIES * 128,), jnp.float32),
            pltpu.VMEM((RPW,), jnp.int32),
            pltpu.VMEM((RPW,), jnp.float32),
            pltpu.VMEM((RPW,), jnp.float32),
            pltpu.VMEM((RPW,), jnp.int32),
            pltpu.SemaphoreType.DMA,
            pltpu.SemaphoreType.DMA,
            pltpu.SemaphoreType.DMA,
            pltpu.SemaphoreType.DMA,
        ],
    )
    return run(packed_words, species_ids, tbl.reshape(-1))
